# baseline (speedup 1.0000x reference)
import jax
import jax.numpy as jnp
from jax import lax
from jax.experimental import pallas as pl
from jax.experimental.pallas import tpu as pltpu
from jax.experimental.pallas import tpu_sc as plsc

HALF = 384
B, L = 1024, 50
NC, NS = 2, 16
NW = NC * NS
BB = B // 128
CPW = 2 * L * BB // NW


def _sc_gather(table, tok):
    mesh = plsc.VectorSubcoreMesh(core_axis_name="c", subcore_axis_name="s")

    @pl.kernel(
        out_type=jax.ShapeDtypeStruct((L, B, 2 * HALF), jnp.float32),
        mesh=mesh,
        scratch_types=[
            pltpu.VMEM((CPW, 128), jnp.int32),
            pltpu.VMEM((2, 128, HALF), jnp.float32),
            pltpu.SemaphoreType.DMA,
            pltpu.SemaphoreType.DMA,
            pltpu.SemaphoreType.DMA,
            pltpu.SemaphoreType.DMA,
        ],
    )
    def k(table_hbm, tok_hbm, out_hbm, idx_v, buf_v, g0, g1, s0, s1):
        wid = lax.axis_index("c") * NS + lax.axis_index("s")
        half = wid // NS
        c0 = wid * CPW

        pltpu.sync_copy(tok_hbm.at[wid], idx_v)
        bias = jnp.full((16,), half * 512, dtype=jnp.int32)

        def bias_row(r):
            for q in range(8):
                sl = pl.ds(q * 16, 16)
                idx_v[r, sl] = idx_v[r, sl] + bias

        gsem = (g0, g1)
        ssem = (s0, s1)

        def start_gather(i, bb):
            pltpu.async_copy(table_hbm.at[idx_v.at[i]], buf_v.at[bb],
                             gsem[bb])

        def start_scatter(i, bb):
            r = c0 + i - half * (L * BB)
            l = r // BB
            blk = lax.rem(r, BB)
            return pltpu.async_copy(
                buf_v.at[bb],
                out_hbm.at[l, pl.ds(blk * 128, 128),
                           pl.ds(half * HALF, HALF)],
                ssem[bb])

        def wait_gather(bb):
            pltpu.make_async_copy(table_hbm.at[pl.ds(0, 128)],
                                  buf_v.at[bb], gsem[bb]).wait()

        def wait_scatter(bb):
            pltpu.make_async_copy(
                buf_v.at[bb],
                out_hbm.at[0, pl.ds(0, 128), pl.ds(0, HALF)],
                ssem[bb]).wait()

        bias_row(0)
        bias_row(1)
        start_gather(0, 0)
        start_gather(1, 1)

        @pl.loop(2, CPW)
        def _bias(r):
            bias_row(r)

        wait_gather(0)
        start_scatter(0, 0)
        wait_scatter(0)
        start_gather(2, 0)
        wait_gather(1)
        start_scatter(1, 1)

        @pl.loop(2, CPW - 1, step=2)
        def _pipe(base):
            for t in range(2):
                i = base + t
                wait_scatter(1 - t)
                start_gather(i + 1, 1 - t)
                wait_gather(t)
                start_scatter(i, t)

        wait_scatter(1)
        wait_gather(0)
        start_scatter(CPW - 1, 0)
        wait_scatter(0)

    return k(table, tok)


def kernel(token, row_embed, col_embed):
    table = jnp.concatenate([row_embed, col_embed], axis=0)
    tok = token.astype(jnp.int32).transpose(2, 1, 0).reshape(NW, CPW, 128)
    out = _sc_gather(table, tok)
    return out.transpose(1, 0, 2)


# speedup vs baseline: 1.0061x; 1.0061x over previous
import jax
import jax.numpy as jnp
from jax import lax
from jax.experimental import pallas as pl
from jax.experimental.pallas import tpu as pltpu
from jax.experimental.pallas import tpu_sc as plsc

HALF = 384
B, L = 1024, 50
NC, NS = 2, 16
NW = NC * NS
BB = B // 128
CPW = 2 * L * BB // NW


def _sc_gather(table, tok):
    mesh = plsc.VectorSubcoreMesh(core_axis_name="c", subcore_axis_name="s")

    @pl.kernel(
        out_type=jax.ShapeDtypeStruct((L, B, 2 * HALF), jnp.float32),
        mesh=mesh,
        scratch_types=[
            pltpu.VMEM((CPW, 128), jnp.int32),
            pltpu.VMEM((2, 128, HALF), jnp.float32),
            pltpu.SemaphoreType.DMA,
            pltpu.SemaphoreType.DMA,
            pltpu.SemaphoreType.DMA,
            pltpu.SemaphoreType.DMA,
        ],
    )
    def k(table_hbm, tok_hbm, out_hbm, idx_v, buf_v, g0, g1, s0, s1):
        wid = lax.axis_index("c") * NS + lax.axis_index("s")
        half = wid // NS
        c0 = wid * CPW

        pltpu.sync_copy(tok_hbm.at[wid], idx_v)
        bias = jnp.full((16,), half * 512, dtype=jnp.int32)

        @pl.loop(0, CPW)
        def _bias(r):
            for q in range(8):
                sl = pl.ds(q * 16, 16)
                idx_v[r, sl] = idx_v[r, sl] + bias

        gsem = (g0, g1)
        ssem = (s0, s1)

        def start_gather(i, bb):
            pltpu.async_copy(table_hbm.at[idx_v.at[i]], buf_v.at[bb],
                             gsem[bb])

        def start_scatter(i, bb):
            r = c0 + i - half * (L * BB)
            l = r // BB
            blk = lax.rem(r, BB)
            return pltpu.async_copy(
                buf_v.at[bb],
                out_hbm.at[l, pl.ds(blk * 128, 128),
                           pl.ds(half * HALF, HALF)],
                ssem[bb])

        def wait_gather(bb):
            pltpu.make_async_copy(table_hbm.at[pl.ds(0, 128)],
                                  buf_v.at[bb], gsem[bb]).wait()

        def wait_scatter(bb):
            pltpu.make_async_copy(
                buf_v.at[bb],
                out_hbm.at[0, pl.ds(0, 128), pl.ds(0, HALF)],
                ssem[bb]).wait()

        start_gather(0, 0)
        start_gather(1, 1)
        wait_gather(0)
        start_scatter(0, 0)
        wait_scatter(0)
        start_gather(2, 0)
        wait_gather(1)
        start_scatter(1, 1)

        @pl.loop(2, CPW - 1, step=2)
        def _pipe(base):
            for t in range(2):
                i = base + t
                wait_scatter(1 - t)
                start_gather(i + 1, 1 - t)
                wait_gather(t)
                start_scatter(i, t)

        wait_scatter(1)
        wait_gather(0)
        start_scatter(CPW - 1, 0)
        wait_scatter(0)

    return k(table, tok)


def kernel(token, row_embed, col_embed):
    table = jnp.concatenate([row_embed, col_embed], axis=0)
    tok = token.astype(jnp.int32).transpose(2, 1, 0).reshape(NW, CPW, 128)
    out = _sc_gather(table, tok)
    return out.transpose(1, 0, 2)
